# baseline (speedup 1.0000x reference)
import functools

import jax
import jax.numpy as jnp
from jax import lax
from jax.experimental import pallas as pl
from jax.experimental.pallas import tpu as pltpu
from jax.experimental.pallas import tpu_sc as plsc

T = 8192
D = 768
F = 1024
E = 8
K = 2

TB = 1024
TM = 512
M = T * K
NT = M // TM

NC = 2
NS = 16
NW = NC * NS
TPW = T // NW
CT = 64
CT2 = 16


def _router_body(gl_ref, i1_ref, i2_ref, wa_ref, wb_ref,
                 p1_ref, p2_ref, cnt_ref, off_ref, carry_ref):
    b = pl.program_id(0)

    @pl.when(b == 0)
    def _():
        carry_ref[...] = jnp.zeros_like(carry_ref)

    logits = gl_ref[...]
    eids = lax.broadcasted_iota(jnp.int32, (TB, E), 1)
    m1 = jnp.max(logits, axis=1, keepdims=True)
    i1 = jnp.min(jnp.where(logits == m1, eids, E), axis=1)[:, None]
    oh1 = (eids == i1).astype(jnp.float32)
    masked = jnp.where(eids == i1, -jnp.inf, logits)
    m2 = jnp.max(masked, axis=1, keepdims=True)
    i2 = jnp.min(jnp.where(masked == m2, eids, E), axis=1)[:, None]
    oh2 = (eids == i2).astype(jnp.float32)

    t = jnp.exp(m2 - m1)
    wa = 1.0 / (1.0 + t)
    wb = 1.0 - wa

    ohs = oh1 + oh2
    r_i = lax.broadcasted_iota(jnp.int32, (TB, TB), 0)
    c_i = lax.broadcasted_iota(jnp.int32, (TB, TB), 1)
    ltri = (c_i < r_i).astype(jnp.float32)
    s_excl = lax.dot_general(ltri, ohs, (((1,), (0,)), ((), ())),
                             preferred_element_type=jnp.float32)
    base = carry_ref[...] + s_excl
    p1 = jnp.sum(oh1 * base, axis=1)[:, None]
    p2 = jnp.sum(oh2 * (base + oh1), axis=1)[:, None]
    new_carry = carry_ref[...][-1:] + jnp.sum(ohs, axis=0, keepdims=True)
    carry_ref[...] = jnp.broadcast_to(new_carry, (TB, E))

    i1_ref[...] = i1
    i2_ref[...] = i2
    wa_ref[...] = wa
    wb_ref[...] = wb
    p1_ref[...] = p1.astype(jnp.int32)
    p2_ref[...] = p2.astype(jnp.int32)
    cnt_ref[...] = new_carry.astype(jnp.int32)
    s = new_carry
    s = s + jnp.concatenate([jnp.zeros((1, 1), s.dtype), s[:, :-1]], axis=1)
    s = s + jnp.concatenate([jnp.zeros((1, 2), s.dtype), s[:, :-2]], axis=1)
    s = s + jnp.concatenate([jnp.zeros((1, 4), s.dtype), s[:, :-4]], axis=1)
    off_ref[...] = (s - new_carry).astype(jnp.int32)


def _router(gl):
    nb = T // TB
    out_shapes = (
        jax.ShapeDtypeStruct((T, 1), jnp.int32),
        jax.ShapeDtypeStruct((T, 1), jnp.int32),
        jax.ShapeDtypeStruct((T, 1), jnp.float32),
        jax.ShapeDtypeStruct((T, 1), jnp.float32),
        jax.ShapeDtypeStruct((T, 1), jnp.int32),
        jax.ShapeDtypeStruct((T, 1), jnp.int32),
        jax.ShapeDtypeStruct((1, E), jnp.int32),
        jax.ShapeDtypeStruct((1, E), jnp.int32),
    )
    tok_spec = pl.BlockSpec((TB, 1), lambda b: (b, 0))
    return pl.pallas_call(
        _router_body,
        grid=(nb,),
        in_specs=[
            pl.BlockSpec((TB, E), lambda b: (b, 0)),
        ],
        out_specs=(tok_spec, tok_spec, tok_spec, tok_spec, tok_spec, tok_spec,
                   pl.BlockSpec((1, E), lambda b: (0, 0)),
                   pl.BlockSpec((1, E), lambda b: (0, 0))),
        out_shape=out_shapes,
        scratch_shapes=[pltpu.VMEM((TB, E), jnp.float32)],
    )(gl)


def _dispatch(x, i1, i2, p1, p2, offsets):
    mesh = plsc.VectorSubcoreMesh(core_axis_name="c", subcore_axis_name="s")

    nchunk = TPW // CT

    @functools.partial(
        pl.kernel,
        mesh=mesh,
        out_type=jax.ShapeDtypeStruct((M, D), jnp.float32),
        scratch_types=[
            pltpu.VMEM((CT, D), jnp.float32),
            pltpu.VMEM((CT, D), jnp.float32),
            pltpu.VMEM((CT,), jnp.int32),
            pltpu.VMEM((CT,), jnp.int32),
            pltpu.VMEM((CT,), jnp.int32),
            pltpu.VMEM((CT,), jnp.int32),
            pltpu.VMEM((CT,), jnp.int32),
            pltpu.VMEM((16,), jnp.int32),
            pltpu.SemaphoreType.DMA,
            pltpu.SemaphoreType.DMA,
            pltpu.SemaphoreType.DMA,
        ],
        compiler_params=pltpu.CompilerParams(needs_layout_passes=False),
    )
    def k(x_hbm, i1_hbm, i2_hbm, p1_hbm, p2_hbm, off_hbm, xs_hbm,
          xba, xbb, q1a, q2a, q1b, q2b, eb, offs, semx, sema, semb):
        wid = lax.axis_index("s") * NC + lax.axis_index("c")
        offs[...] = jnp.zeros((16,), jnp.int32)
        pltpu.sync_copy(off_hbm, offs.at[pl.ds(0, E)])

        xb = (xba, xbb)
        q1 = (q1a, q1b)
        q2 = (q2a, q2b)
        sems = (sema, semb)

        def load_idx(c, b):
            tok0 = wid * TPW + c * CT
            pltpu.sync_copy(i1_hbm.at[pl.ds(tok0, CT)], eb)
            pltpu.sync_copy(p1_hbm.at[pl.ds(tok0, CT)], q1[b])
            for v in range(CT // 16):
                sl = pl.ds(v * 16, 16)
                q1[b][sl] = q1[b][sl] + plsc.load_gather(offs, [eb[sl]])
            pltpu.sync_copy(i2_hbm.at[pl.ds(tok0, CT)], eb)
            pltpu.sync_copy(p2_hbm.at[pl.ds(tok0, CT)], q2[b])
            for v in range(CT // 16):
                sl = pl.ds(v * 16, 16)
                q2[b][sl] = q2[b][sl] + plsc.load_gather(offs, [eb[sl]])

        cpx = pltpu.async_copy(x_hbm.at[pl.ds(wid * TPW, CT)], xba, semx)
        load_idx(0, 0)
        cpx.wait()
        for c in range(nchunk):
            b = c % 2
            nb = 1 - b
            if c + 1 < nchunk:
                tok0n = wid * TPW + (c + 1) * CT
                cpx = pltpu.async_copy(x_hbm.at[pl.ds(tok0n, CT)], xb[nb],
                                       semx)
            s1 = pltpu.async_copy(xb[b], xs_hbm.at[q1[b]], sems[b])
            s2 = pltpu.async_copy(xb[b], xs_hbm.at[q2[b]], sems[b])
            if c + 1 < nchunk:
                load_idx(c + 1, nb)
                cpx.wait()
            s1.wait()
            s2.wait()

    return k(x, i1, i2, p1, p2, offsets)


NS_ITEMS = NT + E - 1


def _experts_body(tl_ref, sel_ref, act_ref, rs_ref, re_ref, wn_ref,
                  xs_ref, w1_ref, w3_ref, w2_ref, out_ref,
                  w1b, w3b, w2b):
    s = pl.program_id(0)

    @pl.when(wn_ref[s] == 1)
    def _():
        w1b[...] = w1_ref[0].astype(jnp.bfloat16)
        w3b[...] = w3_ref[0].astype(jnp.bfloat16)
        w2b[...] = w2_ref[0].astype(jnp.bfloat16)

    @pl.when(act_ref[s] == 1)
    def _():
        xb = xs_ref[...].astype(jnp.bfloat16)
        a = lax.dot_general(xb, w1b[...], (((1,), (1,)), ((), ())),
                            preferred_element_type=jnp.float32)
        g = lax.dot_general(xb, w3b[...], (((1,), (1,)), ((), ())),
                            preferred_element_type=jnp.float32)
        h = ((a / (1.0 + jnp.exp(-a))) * g).astype(jnp.bfloat16)
        y = lax.dot_general(h, w2b[...], (((1,), (0,)), ((), ())),
                            preferred_element_type=jnp.float32)
        rows = lax.broadcasted_iota(jnp.int32, (TM, 1), 0)
        mask = (rows >= rs_ref[s]) & (rows < re_ref[s])
        out_ref[...] = jnp.where(mask, y, out_ref[...])


def _experts(xs, w1, w2, w3, counts):
    cnts = counts.reshape(E)
    off = jnp.concatenate([jnp.zeros((1,), jnp.int32), jnp.cumsum(cnts)])
    t0 = jnp.arange(NT, dtype=jnp.int32) * TM
    sg = jnp.maximum(off[:E][None, :], t0[:, None])
    eg = jnp.minimum(off[1:][None, :], t0[:, None] + TM)
    act = (eg > sg).astype(jnp.int32)
    rsf = (sg - t0[:, None]).reshape(-1)
    ref_ = (eg - t0[:, None]).reshape(-1)
    afl = act.reshape(-1)
    csum = jnp.cumsum(afl)
    n_items = csum[-1]
    k = jnp.arange(NS_ITEMS, dtype=jnp.int32)
    target = jnp.minimum(k + 1, n_items)
    s_k = jnp.searchsorted(csum, target, side="left").astype(jnp.int32)
    tl = s_k // E
    sel = s_k % E
    act_k = (k < n_items).astype(jnp.int32)
    rs_k = jnp.where(act_k == 1, rsf[s_k], 0).astype(jnp.int32)
    re_k = jnp.where(act_k == 1, ref_[s_k], 0).astype(jnp.int32)
    wn_k = jnp.where(
        k == 0, 1,
        (sel != jnp.roll(sel, 1)).astype(jnp.int32)).astype(jnp.int32)

    grid_spec = pltpu.PrefetchScalarGridSpec(
        num_scalar_prefetch=6,
        grid=(NS_ITEMS,),
        in_specs=[
            pl.BlockSpec((TM, D),
                         lambda s, tl, sl, a, r1, r2, wn: (tl[s], 0)),
            pl.BlockSpec((1, F, D),
                         lambda s, tl, sl, a, r1, r2, wn: (sl[s], 0, 0)),
            pl.BlockSpec((1, F, D),
                         lambda s, tl, sl, a, r1, r2, wn: (sl[s], 0, 0)),
            pl.BlockSpec((1, F, D),
                         lambda s, tl, sl, a, r1, r2, wn: (sl[s], 0, 0)),
        ],
        out_specs=pl.BlockSpec((TM, D),
                               lambda s, tl, sl, a, r1, r2, wn: (tl[s], 0)),
        scratch_shapes=[
            pltpu.VMEM((F, D), jnp.bfloat16),
            pltpu.VMEM((F, D), jnp.bfloat16),
            pltpu.VMEM((F, D), jnp.bfloat16),
        ],
    )
    return pl.pallas_call(
        _experts_body,
        grid_spec=grid_spec,
        out_shape=jax.ShapeDtypeStruct((M, D), jnp.float32),
    )(tl, sel, act_k, rs_k, re_k, wn_k, xs, w1, w3, w2)


def _combine(ys, p1, p2, wa, wb, i1, i2, offsets):
    mesh = plsc.VectorSubcoreMesh(core_axis_name="c", subcore_axis_name="s")

    nchunk = TPW // CT2

    @functools.partial(
        pl.kernel,
        mesh=mesh,
        out_type=jax.ShapeDtypeStruct((T, D), jnp.float32),
        scratch_types=[
            pltpu.VMEM((CT2, D), jnp.float32),
            pltpu.VMEM((CT2, D), jnp.float32),
            pltpu.VMEM((CT2, D), jnp.float32),
            pltpu.VMEM((CT2, D), jnp.float32),
            pltpu.VMEM((CT2, D), jnp.float32),
            pltpu.VMEM((CT2, D), jnp.float32),
            pltpu.VMEM((CT2,), jnp.int32),
            pltpu.VMEM((CT2,), jnp.int32),
            pltpu.VMEM((CT2,), jnp.int32),
            pltpu.VMEM((CT2,), jnp.int32),
            pltpu.VMEM((CT2,), jnp.int32),
            pltpu.VMEM((CT2,), jnp.float32),
            pltpu.VMEM((CT2,), jnp.float32),
            pltpu.VMEM((CT2,), jnp.float32),
            pltpu.VMEM((CT2,), jnp.float32),
            pltpu.VMEM((16,), jnp.int32),
            pltpu.SemaphoreType.DMA,
            pltpu.SemaphoreType.DMA,
            pltpu.SemaphoreType.DMA,
        ],
        compiler_params=pltpu.CompilerParams(needs_layout_passes=False),
    )
    def k(ys_hbm, p1_hbm, p2_hbm, wa_hbm, wb_hbm, i1_hbm, i2_hbm, off_hbm,
          out_hbm, aba, abb, bba, bbb, oba, obb, q1a, q1b, q2a, q2b, eb,
          vaa, vab, vba, vbb, offs, sga, sgb, sout):
        wid = lax.axis_index("s") * NC + lax.axis_index("c")
        offs[...] = jnp.zeros((16,), jnp.int32)
        pltpu.sync_copy(off_hbm, offs.at[pl.ds(0, E)])

        ab = (aba, abb)
        bb = (bba, bbb)
        ob = (oba, obb)
        q1 = (q1a, q1b)
        q2 = (q2a, q2b)
        va = (vaa, vab)
        vb = (vba, vbb)
        sg = (sga, sgb)

        def load_idx(c, b):
            tok0 = wid * TPW + c * CT2
            pltpu.sync_copy(p1_hbm.at[pl.ds(tok0, CT2)], q1[b])
            pltpu.sync_copy(i1_hbm.at[pl.ds(tok0, CT2)], eb)
            for v in range(CT2 // 16):
                sl = pl.ds(v * 16, 16)
                q1[b][sl] = q1[b][sl] + plsc.load_gather(offs, [eb[sl]])
            pltpu.sync_copy(p2_hbm.at[pl.ds(tok0, CT2)], q2[b])
            pltpu.sync_copy(i2_hbm.at[pl.ds(tok0, CT2)], eb)
            for v in range(CT2 // 16):
                sl = pl.ds(v * 16, 16)
                q2[b][sl] = q2[b][sl] + plsc.load_gather(offs, [eb[sl]])
            pltpu.sync_copy(wa_hbm.at[pl.ds(tok0, CT2)], va[b])
            pltpu.sync_copy(wb_hbm.at[pl.ds(tok0, CT2)], vb[b])

        def start_gathers(b):
            ca = pltpu.async_copy(ys_hbm.at[q1[b]], ab[b], sg[b])
            cbv = pltpu.async_copy(ys_hbm.at[q2[b]], bb[b], sg[b])
            return ca, cbv

        load_idx(0, 0)
        cps = start_gathers(0)
        pend = {0: cps}
        ost = [None, None]
        for c in range(nchunk):
            b = c % 2
            nb = 1 - b
            if c + 1 < nchunk:
                load_idx(c + 1, nb)
            ca, cbv = pend.pop(c)
            ca.wait()
            cbv.wait()
            if c + 1 < nchunk:
                pend[c + 1] = start_gathers(nb)
            if ost[b] is not None:
                ost[b].wait()

            def tok(j, _, _b=b):
                j16 = jnp.full((16,), j, jnp.int32)
                wa16 = plsc.load_gather(va[_b], [j16])
                wb16 = plsc.load_gather(vb[_b], [j16])
                for cc in range(D // 16):
                    sl = pl.ds(cc * 16, 16)
                    ob[_b][j, sl] = (wa16 * ab[_b][j, sl]
                                     + wb16 * bb[_b][j, sl])
                return 0

            lax.fori_loop(0, CT2, tok, 0)
            tok0 = wid * TPW + c * CT2
            ost[b] = pltpu.async_copy(ob[b], out_hbm.at[pl.ds(tok0, CT2)],
                                      sout)
        ost[(nchunk - 1) % 2].wait()
        if nchunk > 1:
            ost[nchunk % 2].wait()

    return k(ys, p1, p2, wa, wb, i1, i2, offsets)


@jax.jit
def kernel(x, gate_w, w1, w2, w3):
    gl = x @ gate_w.T
    i1, i2, wa, wb, p1, p2, counts, offs = _router(gl)
    i1f = i1.reshape(T)
    i2f = i2.reshape(T)
    p1f = p1.reshape(T)
    p2f = p2.reshape(T)
    offf = offs.reshape(E)
    xs = _dispatch(x, i1f, i2f, p1f, p2f, offf)
    ys = _experts(xs, w1, w2, w3, counts)
    out = _combine(ys, p1f, p2f, wa.reshape(T), wb.reshape(T), i1f, i2f, offf)
    return out


# speedup vs baseline: 1.0762x; 1.0762x over previous
import functools

import jax
import jax.numpy as jnp
from jax import lax
from jax.experimental import pallas as pl
from jax.experimental.pallas import tpu as pltpu
from jax.experimental.pallas import tpu_sc as plsc

T = 8192
D = 768
F = 1024
E = 8
K = 2

TB = 1024
TM = 512
M = T * K
NT = M // TM

NC = 2
NS = 16
NW = NC * NS
TPW = T // NW
CT = 64
CT2 = 32


def _router_body(gl_ref, i1_ref, i2_ref, wa_ref, wb_ref,
                 p1_ref, p2_ref, cnt_ref, off_ref, carry_ref):
    b = pl.program_id(0)

    @pl.when(b == 0)
    def _():
        carry_ref[...] = jnp.zeros_like(carry_ref)

    logits = gl_ref[...]
    eids = lax.broadcasted_iota(jnp.int32, (TB, E), 1)
    m1 = jnp.max(logits, axis=1, keepdims=True)
    i1 = jnp.min(jnp.where(logits == m1, eids, E), axis=1)[:, None]
    oh1 = (eids == i1).astype(jnp.float32)
    masked = jnp.where(eids == i1, -jnp.inf, logits)
    m2 = jnp.max(masked, axis=1, keepdims=True)
    i2 = jnp.min(jnp.where(masked == m2, eids, E), axis=1)[:, None]
    oh2 = (eids == i2).astype(jnp.float32)

    t = jnp.exp(m2 - m1)
    wa = 1.0 / (1.0 + t)
    wb = 1.0 - wa

    ohs = oh1 + oh2
    r_i = lax.broadcasted_iota(jnp.int32, (TB, TB), 0)
    c_i = lax.broadcasted_iota(jnp.int32, (TB, TB), 1)
    ltri = (c_i < r_i).astype(jnp.float32)
    s_excl = lax.dot_general(ltri, ohs, (((1,), (0,)), ((), ())),
                             preferred_element_type=jnp.float32)
    base = carry_ref[...] + s_excl
    p1 = jnp.sum(oh1 * base, axis=1)[:, None]
    p2 = jnp.sum(oh2 * (base + oh1), axis=1)[:, None]
    new_carry = carry_ref[...][-1:] + jnp.sum(ohs, axis=0, keepdims=True)
    carry_ref[...] = jnp.broadcast_to(new_carry, (TB, E))

    i1_ref[...] = i1
    i2_ref[...] = i2
    wa_ref[...] = wa
    wb_ref[...] = wb
    p1_ref[...] = p1.astype(jnp.int32)
    p2_ref[...] = p2.astype(jnp.int32)
    cnt_ref[...] = new_carry.astype(jnp.int32)
    s = new_carry
    s = s + jnp.concatenate([jnp.zeros((1, 1), s.dtype), s[:, :-1]], axis=1)
    s = s + jnp.concatenate([jnp.zeros((1, 2), s.dtype), s[:, :-2]], axis=1)
    s = s + jnp.concatenate([jnp.zeros((1, 4), s.dtype), s[:, :-4]], axis=1)
    off_ref[...] = (s - new_carry).astype(jnp.int32)


def _router(gl):
    nb = T // TB
    out_shapes = (
        jax.ShapeDtypeStruct((T, 1), jnp.int32),
        jax.ShapeDtypeStruct((T, 1), jnp.int32),
        jax.ShapeDtypeStruct((T, 1), jnp.float32),
        jax.ShapeDtypeStruct((T, 1), jnp.float32),
        jax.ShapeDtypeStruct((T, 1), jnp.int32),
        jax.ShapeDtypeStruct((T, 1), jnp.int32),
        jax.ShapeDtypeStruct((1, E), jnp.int32),
        jax.ShapeDtypeStruct((1, E), jnp.int32),
    )
    tok_spec = pl.BlockSpec((TB, 1), lambda b: (b, 0))
    return pl.pallas_call(
        _router_body,
        grid=(nb,),
        in_specs=[
            pl.BlockSpec((TB, E), lambda b: (b, 0)),
        ],
        out_specs=(tok_spec, tok_spec, tok_spec, tok_spec, tok_spec, tok_spec,
                   pl.BlockSpec((1, E), lambda b: (0, 0)),
                   pl.BlockSpec((1, E), lambda b: (0, 0))),
        out_shape=out_shapes,
        scratch_shapes=[pltpu.VMEM((TB, E), jnp.float32)],
    )(gl)


def _dispatch(x, i1, i2, p1, p2, offsets):
    mesh = plsc.VectorSubcoreMesh(core_axis_name="c", subcore_axis_name="s")

    @functools.partial(
        pl.kernel,
        mesh=mesh,
        out_type=jax.ShapeDtypeStruct((M, D), jnp.float32),
        scratch_types=[
            pltpu.VMEM((CT, D), jnp.float32),
            pltpu.VMEM((CT,), jnp.int32),
            pltpu.VMEM((CT,), jnp.int32),
            pltpu.VMEM((16,), jnp.int32),
            pltpu.SemaphoreType.DMA,
            pltpu.SemaphoreType.DMA,
        ],
        compiler_params=pltpu.CompilerParams(needs_layout_passes=False),
    )
    def k(x_hbm, i1_hbm, i2_hbm, p1_hbm, p2_hbm, off_hbm, xs_hbm,
          xb, q1, q2, offs, sem1, sem2):
        wid = lax.axis_index("s") * NC + lax.axis_index("c")
        offs[...] = jnp.zeros((16,), jnp.int32)
        pltpu.sync_copy(off_hbm, offs.at[pl.ds(0, E)])

        for c in range(TPW // CT):
            tok0 = wid * TPW + c * CT
            pltpu.sync_copy(x_hbm.at[pl.ds(tok0, CT)], xb)
            pltpu.sync_copy(i1_hbm.at[pl.ds(tok0, CT)], q1)
            pltpu.sync_copy(p1_hbm.at[pl.ds(tok0, CT)], q2)
            for v in range(CT // 16):
                sl = pl.ds(v * 16, 16)
                q2[sl] = q2[sl] + plsc.load_gather(offs, [q1[sl]])
            cp1 = pltpu.async_copy(xb, xs_hbm.at[q2], sem1)
            pltpu.sync_copy(i2_hbm.at[pl.ds(tok0, CT)], q1)
            cp1.wait()
            pltpu.sync_copy(p2_hbm.at[pl.ds(tok0, CT)], q2)
            for v in range(CT // 16):
                sl = pl.ds(v * 16, 16)
                q2[sl] = q2[sl] + plsc.load_gather(offs, [q1[sl]])
            pltpu.async_copy(xb, xs_hbm.at[q2], sem2).wait()

    return k(x, i1, i2, p1, p2, offsets)


NS_ITEMS = NT + E - 1


def _experts_body(tl_ref, sel_ref, act_ref, rs_ref, re_ref, wn_ref,
                  xs_ref, w1_ref, w3_ref, w2_ref, out_ref):
    s = pl.program_id(0)

    @pl.when(act_ref[s] == 1)
    def _():
        xb = xs_ref[...].astype(jnp.bfloat16)
        a = lax.dot_general(xb, w1_ref[0].astype(jnp.bfloat16),
                            (((1,), (1,)), ((), ())),
                            preferred_element_type=jnp.float32)
        g = lax.dot_general(xb, w3_ref[0].astype(jnp.bfloat16),
                            (((1,), (1,)), ((), ())),
                            preferred_element_type=jnp.float32)
        h = ((a / (1.0 + jnp.exp(-a))) * g).astype(jnp.bfloat16)
        y = lax.dot_general(h, w2_ref[0].astype(jnp.bfloat16),
                            (((1,), (0,)), ((), ())),
                            preferred_element_type=jnp.float32)
        rows = lax.broadcasted_iota(jnp.int32, (TM, 1), 0)
        mask = (rows >= rs_ref[s]) & (rows < re_ref[s])
        out_ref[...] = jnp.where(mask, y, out_ref[...])


def _experts(xs, w1, w2, w3, counts):
    cnts = counts.reshape(E)
    off = jnp.concatenate([jnp.zeros((1,), jnp.int32), jnp.cumsum(cnts)])
    t0 = jnp.arange(NT, dtype=jnp.int32) * TM
    sg = jnp.maximum(off[:E][None, :], t0[:, None])
    eg = jnp.minimum(off[1:][None, :], t0[:, None] + TM)
    act = (eg > sg).astype(jnp.int32)
    rsf = (sg - t0[:, None]).reshape(-1)
    ref_ = (eg - t0[:, None]).reshape(-1)
    afl = act.reshape(-1)
    csum = jnp.cumsum(afl)
    n_items = csum[-1]
    k = jnp.arange(NS_ITEMS, dtype=jnp.int32)
    target = jnp.minimum(k + 1, n_items)
    s_k = jnp.searchsorted(csum, target, side="left").astype(jnp.int32)
    tl = s_k // E
    sel = s_k % E
    act_k = (k < n_items).astype(jnp.int32)
    rs_k = jnp.where(act_k == 1, rsf[s_k], 0).astype(jnp.int32)
    re_k = jnp.where(act_k == 1, ref_[s_k], 0).astype(jnp.int32)
    wn_k = jnp.where(
        k == 0, 1,
        (sel != jnp.roll(sel, 1)).astype(jnp.int32)).astype(jnp.int32)

    grid_spec = pltpu.PrefetchScalarGridSpec(
        num_scalar_prefetch=6,
        grid=(NS_ITEMS,),
        in_specs=[
            pl.BlockSpec((TM, D),
                         lambda s, tl, sl, a, r1, r2, wn: (tl[s], 0)),
            pl.BlockSpec((1, F, D),
                         lambda s, tl, sl, a, r1, r2, wn: (sl[s], 0, 0)),
            pl.BlockSpec((1, F, D),
                         lambda s, tl, sl, a, r1, r2, wn: (sl[s], 0, 0)),
            pl.BlockSpec((1, F, D),
                         lambda s, tl, sl, a, r1, r2, wn: (sl[s], 0, 0)),
        ],
        out_specs=pl.BlockSpec((TM, D),
                               lambda s, tl, sl, a, r1, r2, wn: (tl[s], 0)),
    )
    return pl.pallas_call(
        _experts_body,
        grid_spec=grid_spec,
        out_shape=jax.ShapeDtypeStruct((M, D), jnp.float32),
    )(tl, sel, act_k, rs_k, re_k, wn_k, xs, w1, w3, w2)


def _combine(ys, p1, p2, wa, wb, i1, i2, offsets):
    mesh = plsc.VectorSubcoreMesh(core_axis_name="c", subcore_axis_name="s")

    @functools.partial(
        pl.kernel,
        mesh=mesh,
        out_type=jax.ShapeDtypeStruct((T, D), jnp.float32),
        scratch_types=[
            pltpu.VMEM((CT2, D), jnp.float32),
            pltpu.VMEM((CT2, D), jnp.float32),
            pltpu.VMEM((CT2, D), jnp.float32),
            pltpu.VMEM((CT2,), jnp.int32),
            pltpu.VMEM((CT2,), jnp.int32),
            pltpu.VMEM((CT2,), jnp.int32),
            pltpu.VMEM((CT2,), jnp.int32),
            pltpu.VMEM((CT2,), jnp.float32),
            pltpu.VMEM((CT2,), jnp.float32),
            pltpu.VMEM((16,), jnp.int32),
            pltpu.SemaphoreType.DMA,
            pltpu.SemaphoreType.DMA,
        ],
        compiler_params=pltpu.CompilerParams(needs_layout_passes=False),
    )
    def k(ys_hbm, p1_hbm, p2_hbm, wa_hbm, wb_hbm, i1_hbm, i2_hbm, off_hbm,
          out_hbm, ab, bb, ob, q1, q2, e1, e2, va, vb, offs, sem1, sem2):
        wid = lax.axis_index("s") * NC + lax.axis_index("c")
        offs[...] = jnp.zeros((16,), jnp.int32)
        pltpu.sync_copy(off_hbm, offs.at[pl.ds(0, E)])

        def chunk(c, _):
            tok0 = wid * TPW + c * CT2
            pltpu.sync_copy(p1_hbm.at[pl.ds(tok0, CT2)], q1)
            pltpu.sync_copy(i1_hbm.at[pl.ds(tok0, CT2)], e1)
            for v in range(CT2 // 16):
                sl = pl.ds(v * 16, 16)
                q1[sl] = q1[sl] + plsc.load_gather(offs, [e1[sl]])
            cpa = pltpu.async_copy(ys_hbm.at[q1], ab, sem1)
            pltpu.sync_copy(p2_hbm.at[pl.ds(tok0, CT2)], q2)
            pltpu.sync_copy(i2_hbm.at[pl.ds(tok0, CT2)], e2)
            for v in range(CT2 // 16):
                sl = pl.ds(v * 16, 16)
                q2[sl] = q2[sl] + plsc.load_gather(offs, [e2[sl]])
            cpb = pltpu.async_copy(ys_hbm.at[q2], bb, sem2)
            pltpu.sync_copy(wa_hbm.at[pl.ds(tok0, CT2)], va)
            pltpu.sync_copy(wb_hbm.at[pl.ds(tok0, CT2)], vb)
            cpa.wait()
            cpb.wait()

            def tok(j, _):
                j16 = jnp.full((16,), j, jnp.int32)
                wa16 = plsc.load_gather(va, [j16])
                wb16 = plsc.load_gather(vb, [j16])
                for cc in range(D // 16):
                    sl = pl.ds(cc * 16, 16)
                    ob[j, sl] = wa16 * ab[j, sl] + wb16 * bb[j, sl]
                return 0

            lax.fori_loop(0, CT2, tok, 0)
            pltpu.sync_copy(ob, out_hbm.at[pl.ds(tok0, CT2)])
            return 0

        lax.fori_loop(0, TPW // CT2, chunk, 0)

    return k(ys, p1, p2, wa, wb, i1, i2, offsets)


@jax.jit
def kernel(x, gate_w, w1, w2, w3):
    gl = x @ gate_w.T
    i1, i2, wa, wb, p1, p2, counts, offs = _router(gl)
    i1f = i1.reshape(T)
    i2f = i2.reshape(T)
    p1f = p1.reshape(T)
    p2f = p2.reshape(T)
    offf = offs.reshape(E)
    xs = _dispatch(x, i1f, i2f, p1f, p2f, offf)
    ys = _experts(xs, w1, w2, w3, counts)
    out = _combine(ys, p1f, p2f, wa.reshape(T), wb.reshape(T), i1f, i2f, offf)
    return out


# speedup vs baseline: 1.0905x; 1.0133x over previous
import functools

import jax
import jax.numpy as jnp
from jax import lax
from jax.experimental import pallas as pl
from jax.experimental.pallas import tpu as pltpu
from jax.experimental.pallas import tpu_sc as plsc

T = 8192
D = 768
F = 1024
E = 8
K = 2

TB = 1024
TM = 512
M = T * K
NT = M // TM

NC = 2
NS = 16
NW = NC * NS
TPW = T // NW
CT = 128
CT2 = 32


def _router_body(gl_ref, i1_ref, i2_ref, wa_ref, wb_ref,
                 p1_ref, p2_ref, cnt_ref, off_ref, carry_ref):
    b = pl.program_id(0)

    @pl.when(b == 0)
    def _():
        carry_ref[...] = jnp.zeros_like(carry_ref)

    logits = gl_ref[...]
    eids = lax.broadcasted_iota(jnp.int32, (TB, E), 1)
    m1 = jnp.max(logits, axis=1, keepdims=True)
    i1 = jnp.min(jnp.where(logits == m1, eids, E), axis=1)[:, None]
    oh1 = (eids == i1).astype(jnp.float32)
    masked = jnp.where(eids == i1, -jnp.inf, logits)
    m2 = jnp.max(masked, axis=1, keepdims=True)
    i2 = jnp.min(jnp.where(masked == m2, eids, E), axis=1)[:, None]
    oh2 = (eids == i2).astype(jnp.float32)

    t = jnp.exp(m2 - m1)
    wa = 1.0 / (1.0 + t)
    wb = 1.0 - wa

    ohs = oh1 + oh2
    r_i = lax.broadcasted_iota(jnp.int32, (TB, TB), 0)
    c_i = lax.broadcasted_iota(jnp.int32, (TB, TB), 1)
    ltri = (c_i < r_i).astype(jnp.float32)
    s_excl = lax.dot_general(ltri, ohs, (((1,), (0,)), ((), ())),
                             preferred_element_type=jnp.float32)
    base = carry_ref[...] + s_excl
    p1 = jnp.sum(oh1 * base, axis=1)[:, None]
    p2 = jnp.sum(oh2 * (base + oh1), axis=1)[:, None]
    new_carry = carry_ref[...][-1:] + jnp.sum(ohs, axis=0, keepdims=True)
    carry_ref[...] = jnp.broadcast_to(new_carry, (TB, E))

    i1_ref[...] = i1
    i2_ref[...] = i2
    wa_ref[...] = wa
    wb_ref[...] = wb
    p1_ref[...] = p1.astype(jnp.int32)
    p2_ref[...] = p2.astype(jnp.int32)
    cnt_ref[...] = new_carry.astype(jnp.int32)
    s = new_carry
    s = s + jnp.concatenate([jnp.zeros((1, 1), s.dtype), s[:, :-1]], axis=1)
    s = s + jnp.concatenate([jnp.zeros((1, 2), s.dtype), s[:, :-2]], axis=1)
    s = s + jnp.concatenate([jnp.zeros((1, 4), s.dtype), s[:, :-4]], axis=1)
    off_ref[...] = (s - new_carry).astype(jnp.int32)


def _router(gl):
    nb = T // TB
    out_shapes = (
        jax.ShapeDtypeStruct((T, 1), jnp.int32),
        jax.ShapeDtypeStruct((T, 1), jnp.int32),
        jax.ShapeDtypeStruct((T, 1), jnp.float32),
        jax.ShapeDtypeStruct((T, 1), jnp.float32),
        jax.ShapeDtypeStruct((T, 1), jnp.int32),
        jax.ShapeDtypeStruct((T, 1), jnp.int32),
        jax.ShapeDtypeStruct((1, E), jnp.int32),
        jax.ShapeDtypeStruct((1, E), jnp.int32),
    )
    tok_spec = pl.BlockSpec((TB, 1), lambda b: (b, 0))
    return pl.pallas_call(
        _router_body,
        grid=(nb,),
        in_specs=[
            pl.BlockSpec((TB, E), lambda b: (b, 0)),
        ],
        out_specs=(tok_spec, tok_spec, tok_spec, tok_spec, tok_spec, tok_spec,
                   pl.BlockSpec((1, E), lambda b: (0, 0)),
                   pl.BlockSpec((1, E), lambda b: (0, 0))),
        out_shape=out_shapes,
        scratch_shapes=[pltpu.VMEM((TB, E), jnp.float32)],
    )(gl)


def _dispatch(x, i1, i2, p1, p2, offsets):
    mesh = plsc.VectorSubcoreMesh(core_axis_name="c", subcore_axis_name="s")

    @functools.partial(
        pl.kernel,
        mesh=mesh,
        out_type=jax.ShapeDtypeStruct((M, D), jnp.float32),
        scratch_types=[
            pltpu.VMEM((CT, D), jnp.float32),
            pltpu.VMEM((CT,), jnp.int32),
            pltpu.VMEM((CT,), jnp.int32),
            pltpu.VMEM((16,), jnp.int32),
            pltpu.SemaphoreType.DMA,
            pltpu.SemaphoreType.DMA,
        ],
        compiler_params=pltpu.CompilerParams(needs_layout_passes=False),
    )
    def k(x_hbm, i1_hbm, i2_hbm, p1_hbm, p2_hbm, off_hbm, xs_hbm,
          xb, q1, q2, offs, sem1, sem2):
        wid = lax.axis_index("s") * NC + lax.axis_index("c")
        offs[...] = jnp.zeros((16,), jnp.int32)
        pltpu.sync_copy(off_hbm, offs.at[pl.ds(0, E)])

        for c in range(TPW // CT):
            tok0 = wid * TPW + c * CT
            pltpu.sync_copy(x_hbm.at[pl.ds(tok0, CT)], xb)
            pltpu.sync_copy(i1_hbm.at[pl.ds(tok0, CT)], q1)
            pltpu.sync_copy(p1_hbm.at[pl.ds(tok0, CT)], q2)
            for v in range(CT // 16):
                sl = pl.ds(v * 16, 16)
                q2[sl] = q2[sl] + plsc.load_gather(offs, [q1[sl]])
            cp1 = pltpu.async_copy(xb, xs_hbm.at[q2], sem1)
            pltpu.sync_copy(i2_hbm.at[pl.ds(tok0, CT)], q1)
            cp1.wait()
            pltpu.sync_copy(p2_hbm.at[pl.ds(tok0, CT)], q2)
            for v in range(CT // 16):
                sl = pl.ds(v * 16, 16)
                q2[sl] = q2[sl] + plsc.load_gather(offs, [q1[sl]])
            pltpu.async_copy(xb, xs_hbm.at[q2], sem2).wait()

    return k(x, i1, i2, p1, p2, offsets)


NS_ITEMS = NT + E - 1


def _experts_body(tl_ref, sel_ref, act_ref, rs_ref, re_ref, wn_ref,
                  xs_ref, w1_ref, w3_ref, w2_ref, out_ref):
    s = pl.program_id(0)

    @pl.when(act_ref[s] == 1)
    def _():
        xb = xs_ref[...].astype(jnp.bfloat16)
        a = lax.dot_general(xb, w1_ref[0].astype(jnp.bfloat16),
                            (((1,), (1,)), ((), ())),
                            preferred_element_type=jnp.float32)
        g = lax.dot_general(xb, w3_ref[0].astype(jnp.bfloat16),
                            (((1,), (1,)), ((), ())),
                            preferred_element_type=jnp.float32)
        h = ((a / (1.0 + jnp.exp(-a))) * g).astype(jnp.bfloat16)
        y = lax.dot_general(h, w2_ref[0].astype(jnp.bfloat16),
                            (((1,), (0,)), ((), ())),
                            preferred_element_type=jnp.float32)
        rows = lax.broadcasted_iota(jnp.int32, (TM, 1), 0)
        mask = (rows >= rs_ref[s]) & (rows < re_ref[s])
        out_ref[...] = jnp.where(mask, y, out_ref[...])


def _experts(xs, w1, w2, w3, counts):
    cnts = counts.reshape(E)
    off = jnp.concatenate([jnp.zeros((1,), jnp.int32), jnp.cumsum(cnts)])
    t0 = jnp.arange(NT, dtype=jnp.int32) * TM
    sg = jnp.maximum(off[:E][None, :], t0[:, None])
    eg = jnp.minimum(off[1:][None, :], t0[:, None] + TM)
    act = (eg > sg).astype(jnp.int32)
    rsf = (sg - t0[:, None]).reshape(-1)
    ref_ = (eg - t0[:, None]).reshape(-1)
    afl = act.reshape(-1)
    csum = jnp.cumsum(afl)
    n_items = csum[-1]
    k = jnp.arange(NS_ITEMS, dtype=jnp.int32)
    target = jnp.minimum(k + 1, n_items)
    s_k = jnp.searchsorted(csum, target, side="left").astype(jnp.int32)
    tl = s_k // E
    sel = s_k % E
    act_k = (k < n_items).astype(jnp.int32)
    rs_k = jnp.where(act_k == 1, rsf[s_k], 0).astype(jnp.int32)
    re_k = jnp.where(act_k == 1, ref_[s_k], 0).astype(jnp.int32)
    wn_k = jnp.where(
        k == 0, 1,
        (sel != jnp.roll(sel, 1)).astype(jnp.int32)).astype(jnp.int32)

    grid_spec = pltpu.PrefetchScalarGridSpec(
        num_scalar_prefetch=6,
        grid=(NS_ITEMS,),
        in_specs=[
            pl.BlockSpec((TM, D),
                         lambda s, tl, sl, a, r1, r2, wn: (tl[s], 0)),
            pl.BlockSpec((1, F, D),
                         lambda s, tl, sl, a, r1, r2, wn: (sl[s], 0, 0)),
            pl.BlockSpec((1, F, D),
                         lambda s, tl, sl, a, r1, r2, wn: (sl[s], 0, 0)),
            pl.BlockSpec((1, F, D),
                         lambda s, tl, sl, a, r1, r2, wn: (sl[s], 0, 0)),
        ],
        out_specs=pl.BlockSpec((TM, D),
                               lambda s, tl, sl, a, r1, r2, wn: (tl[s], 0)),
    )
    return pl.pallas_call(
        _experts_body,
        grid_spec=grid_spec,
        out_shape=jax.ShapeDtypeStruct((M, D), jnp.float32),
    )(tl, sel, act_k, rs_k, re_k, wn_k, xs, w1, w3, w2)


def _combine(ys, p1, p2, wa, wb, i1, i2, offsets):
    mesh = plsc.VectorSubcoreMesh(core_axis_name="c", subcore_axis_name="s")

    @functools.partial(
        pl.kernel,
        mesh=mesh,
        out_type=jax.ShapeDtypeStruct((T, D), jnp.float32),
        scratch_types=[
            pltpu.VMEM((CT2, D), jnp.float32),
            pltpu.VMEM((CT2, D), jnp.float32),
            pltpu.VMEM((CT2, D), jnp.float32),
            pltpu.VMEM((CT2,), jnp.int32),
            pltpu.VMEM((CT2,), jnp.int32),
            pltpu.VMEM((CT2,), jnp.int32),
            pltpu.VMEM((CT2,), jnp.int32),
            pltpu.VMEM((CT2,), jnp.float32),
            pltpu.VMEM((CT2,), jnp.float32),
            pltpu.VMEM((16,), jnp.int32),
            pltpu.SemaphoreType.DMA,
            pltpu.SemaphoreType.DMA,
        ],
        compiler_params=pltpu.CompilerParams(needs_layout_passes=False),
    )
    def k(ys_hbm, p1_hbm, p2_hbm, wa_hbm, wb_hbm, i1_hbm, i2_hbm, off_hbm,
          out_hbm, ab, bb, ob, q1, q2, e1, e2, va, vb, offs, sem1, sem2):
        wid = lax.axis_index("s") * NC + lax.axis_index("c")
        offs[...] = jnp.zeros((16,), jnp.int32)
        pltpu.sync_copy(off_hbm, offs.at[pl.ds(0, E)])

        def chunk(c, _):
            tok0 = wid * TPW + c * CT2
            pltpu.sync_copy(p1_hbm.at[pl.ds(tok0, CT2)], q1)
            pltpu.sync_copy(i1_hbm.at[pl.ds(tok0, CT2)], e1)
            for v in range(CT2 // 16):
                sl = pl.ds(v * 16, 16)
                q1[sl] = q1[sl] + plsc.load_gather(offs, [e1[sl]])
            cpa = pltpu.async_copy(ys_hbm.at[q1], ab, sem1)
            pltpu.sync_copy(p2_hbm.at[pl.ds(tok0, CT2)], q2)
            pltpu.sync_copy(i2_hbm.at[pl.ds(tok0, CT2)], e2)
            for v in range(CT2 // 16):
                sl = pl.ds(v * 16, 16)
                q2[sl] = q2[sl] + plsc.load_gather(offs, [e2[sl]])
            cpb = pltpu.async_copy(ys_hbm.at[q2], bb, sem2)
            pltpu.sync_copy(wa_hbm.at[pl.ds(tok0, CT2)], va)
            pltpu.sync_copy(wb_hbm.at[pl.ds(tok0, CT2)], vb)
            cpa.wait()
            cpb.wait()

            def tok(j, _):
                j16 = jnp.full((16,), j, jnp.int32)
                wa16 = plsc.load_gather(va, [j16])
                wb16 = plsc.load_gather(vb, [j16])
                for cc in range(D // 16):
                    sl = pl.ds(cc * 16, 16)
                    ob[j, sl] = wa16 * ab[j, sl] + wb16 * bb[j, sl]
                return 0

            lax.fori_loop(0, CT2, tok, 0)
            pltpu.sync_copy(ob, out_hbm.at[pl.ds(tok0, CT2)])
            return 0

        lax.fori_loop(0, TPW // CT2, chunk, 0)

    return k(ys, p1, p2, wa, wb, i1, i2, offsets)


@jax.jit
def kernel(x, gate_w, w1, w2, w3):
    gl = x @ gate_w.T
    i1, i2, wa, wb, p1, p2, counts, offs = _router(gl)
    i1f = i1.reshape(T)
    i2f = i2.reshape(T)
    p1f = p1.reshape(T)
    p2f = p2.reshape(T)
    offf = offs.reshape(E)
    xs = _dispatch(x, i1f, i2f, p1f, p2f, offf)
    ys = _experts(xs, w1, w2, w3, counts)
    out = _combine(ys, p1f, p2f, wa.reshape(T), wb.reshape(T), i1f, i2f, offf)
    return out


# speedup vs baseline: 1.1068x; 1.0149x over previous
import functools

import jax
import jax.numpy as jnp
from jax import lax
from jax.experimental import pallas as pl
from jax.experimental.pallas import tpu as pltpu
from jax.experimental.pallas import tpu_sc as plsc

T = 8192
D = 768
F = 1024
E = 8
K = 2

TB = 1024
TM = 512
M = T * K
NT = M // TM

NC = 2
NS = 16
NW = NC * NS
TPW = T // NW
CT = 128
CT2 = 32


def _router_body(gl_ref, i1_ref, i2_ref, wa_ref, wb_ref,
                 p1_ref, p2_ref, cnt_ref, off_ref, carry_ref):
    b = pl.program_id(0)

    @pl.when(b == 0)
    def _():
        carry_ref[...] = jnp.zeros_like(carry_ref)

    logits = gl_ref[...]
    eids = lax.broadcasted_iota(jnp.int32, (TB, E), 1)
    m1 = jnp.max(logits, axis=1, keepdims=True)
    i1 = jnp.min(jnp.where(logits == m1, eids, E), axis=1)[:, None]
    oh1 = (eids == i1).astype(jnp.float32)
    masked = jnp.where(eids == i1, -jnp.inf, logits)
    m2 = jnp.max(masked, axis=1, keepdims=True)
    i2 = jnp.min(jnp.where(masked == m2, eids, E), axis=1)[:, None]
    oh2 = (eids == i2).astype(jnp.float32)

    t = jnp.exp(m2 - m1)
    wa = 1.0 / (1.0 + t)
    wb = 1.0 - wa

    ohs = oh1 + oh2
    r_i = lax.broadcasted_iota(jnp.int32, (TB, TB), 0)
    c_i = lax.broadcasted_iota(jnp.int32, (TB, TB), 1)
    ltri = (c_i < r_i).astype(jnp.float32)
    s_excl = lax.dot_general(ltri, ohs, (((1,), (0,)), ((), ())),
                             preferred_element_type=jnp.float32)
    base = carry_ref[...] + s_excl
    p1 = jnp.sum(oh1 * base, axis=1)[:, None]
    p2 = jnp.sum(oh2 * (base + oh1), axis=1)[:, None]
    new_carry = carry_ref[...][-1:] + jnp.sum(ohs, axis=0, keepdims=True)
    carry_ref[...] = jnp.broadcast_to(new_carry, (TB, E))

    i1_ref[...] = i1
    i2_ref[...] = i2
    wa_ref[...] = wa
    wb_ref[...] = wb
    p1_ref[...] = p1.astype(jnp.int32)
    p2_ref[...] = p2.astype(jnp.int32)
    cnt_ref[...] = new_carry.astype(jnp.int32)
    s = new_carry
    s = s + jnp.concatenate([jnp.zeros((1, 1), s.dtype), s[:, :-1]], axis=1)
    s = s + jnp.concatenate([jnp.zeros((1, 2), s.dtype), s[:, :-2]], axis=1)
    s = s + jnp.concatenate([jnp.zeros((1, 4), s.dtype), s[:, :-4]], axis=1)
    off_ref[...] = (s - new_carry).astype(jnp.int32)


def _router(gl):
    nb = T // TB
    out_shapes = (
        jax.ShapeDtypeStruct((T, 1), jnp.int32),
        jax.ShapeDtypeStruct((T, 1), jnp.int32),
        jax.ShapeDtypeStruct((T, 1), jnp.float32),
        jax.ShapeDtypeStruct((T, 1), jnp.float32),
        jax.ShapeDtypeStruct((T, 1), jnp.int32),
        jax.ShapeDtypeStruct((T, 1), jnp.int32),
        jax.ShapeDtypeStruct((1, E), jnp.int32),
        jax.ShapeDtypeStruct((1, E), jnp.int32),
    )
    tok_spec = pl.BlockSpec((TB, 1), lambda b: (b, 0))
    return pl.pallas_call(
        _router_body,
        grid=(nb,),
        in_specs=[
            pl.BlockSpec((TB, E), lambda b: (b, 0)),
        ],
        out_specs=(tok_spec, tok_spec, tok_spec, tok_spec, tok_spec, tok_spec,
                   pl.BlockSpec((1, E), lambda b: (0, 0)),
                   pl.BlockSpec((1, E), lambda b: (0, 0))),
        out_shape=out_shapes,
        scratch_shapes=[pltpu.VMEM((TB, E), jnp.float32)],
    )(gl)


def _dispatch(x, i1, i2, p1, p2, offsets):
    mesh = plsc.VectorSubcoreMesh(core_axis_name="c", subcore_axis_name="s")

    @functools.partial(
        pl.kernel,
        mesh=mesh,
        out_type=jax.ShapeDtypeStruct((M, D), jnp.float32),
        scratch_types=[
            pltpu.VMEM((CT, D), jnp.float32),
            pltpu.VMEM((CT,), jnp.int32),
            pltpu.VMEM((CT,), jnp.int32),
            pltpu.VMEM((16,), jnp.int32),
            pltpu.SemaphoreType.DMA,
            pltpu.SemaphoreType.DMA,
        ],
        compiler_params=pltpu.CompilerParams(needs_layout_passes=False),
    )
    def k(x_hbm, i1_hbm, i2_hbm, p1_hbm, p2_hbm, off_hbm, xs_hbm,
          xb, q1, q2, offs, sem1, sem2):
        wid = lax.axis_index("s") * NC + lax.axis_index("c")
        offs[...] = jnp.zeros((16,), jnp.int32)
        pltpu.sync_copy(off_hbm, offs.at[pl.ds(0, E)])

        for c in range(TPW // CT):
            tok0 = wid * TPW + c * CT
            pltpu.sync_copy(x_hbm.at[pl.ds(tok0, CT)], xb)
            pltpu.sync_copy(i1_hbm.at[pl.ds(tok0, CT)], q1)
            pltpu.sync_copy(p1_hbm.at[pl.ds(tok0, CT)], q2)
            for v in range(CT // 16):
                sl = pl.ds(v * 16, 16)
                q2[sl] = q2[sl] + plsc.load_gather(offs, [q1[sl]])
            cp1 = pltpu.async_copy(xb, xs_hbm.at[q2], sem1)
            pltpu.sync_copy(i2_hbm.at[pl.ds(tok0, CT)], q1)
            cp1.wait()
            pltpu.sync_copy(p2_hbm.at[pl.ds(tok0, CT)], q2)
            for v in range(CT // 16):
                sl = pl.ds(v * 16, 16)
                q2[sl] = q2[sl] + plsc.load_gather(offs, [q1[sl]])
            pltpu.async_copy(xb, xs_hbm.at[q2], sem2).wait()

    return k(x, i1, i2, p1, p2, offsets)


NS_ITEMS = NT + E - 1


def _experts_body(tl_ref, sel_ref, act_ref, rs_ref, re_ref, wn_ref,
                  xs_ref, w1_ref, w3_ref, w2_ref, out_ref):
    s = pl.program_id(0)

    @pl.when(act_ref[s] == 1)
    def _():
        xb = xs_ref[...].astype(jnp.bfloat16)
        a = lax.dot_general(xb, w1_ref[0].astype(jnp.bfloat16),
                            (((1,), (1,)), ((), ())),
                            preferred_element_type=jnp.float32)
        g = lax.dot_general(xb, w3_ref[0].astype(jnp.bfloat16),
                            (((1,), (1,)), ((), ())),
                            preferred_element_type=jnp.float32)
        h = ((a / (1.0 + jnp.exp(-a))) * g).astype(jnp.bfloat16)
        y = lax.dot_general(h, w2_ref[0].astype(jnp.bfloat16),
                            (((1,), (0,)), ((), ())),
                            preferred_element_type=jnp.float32)
        rows = lax.broadcasted_iota(jnp.int32, (TM, 1), 0)
        mask = (rows >= rs_ref[s]) & (rows < re_ref[s])
        out_ref[...] = jnp.where(mask, y, out_ref[...])


def _experts(xs, w1, w2, w3, counts):
    cnts = counts.reshape(E)
    off = jnp.concatenate([jnp.zeros((1,), jnp.int32), jnp.cumsum(cnts)])
    t0 = jnp.arange(NT, dtype=jnp.int32) * TM
    sg = jnp.maximum(off[:E][None, :], t0[:, None])
    eg = jnp.minimum(off[1:][None, :], t0[:, None] + TM)
    act = (eg > sg).astype(jnp.int32)
    rsf = (sg - t0[:, None]).reshape(-1)
    ref_ = (eg - t0[:, None]).reshape(-1)
    afl = act.reshape(-1)
    csum = jnp.cumsum(afl)
    n_items = csum[-1]
    k = jnp.arange(NS_ITEMS, dtype=jnp.int32)
    target = jnp.minimum(k + 1, n_items)
    s_k = jnp.searchsorted(csum, target, side="left").astype(jnp.int32)
    tl = s_k // E
    sel = s_k % E
    act_k = (k < n_items).astype(jnp.int32)
    rs_k = jnp.where(act_k == 1, rsf[s_k], 0).astype(jnp.int32)
    re_k = jnp.where(act_k == 1, ref_[s_k], 0).astype(jnp.int32)
    wn_k = jnp.where(
        k == 0, 1,
        (sel != jnp.roll(sel, 1)).astype(jnp.int32)).astype(jnp.int32)

    grid_spec = pltpu.PrefetchScalarGridSpec(
        num_scalar_prefetch=6,
        grid=(NS_ITEMS,),
        in_specs=[
            pl.BlockSpec((TM, D),
                         lambda s, tl, sl, a, r1, r2, wn: (tl[s], 0)),
            pl.BlockSpec((1, F, D),
                         lambda s, tl, sl, a, r1, r2, wn: (sl[s], 0, 0)),
            pl.BlockSpec((1, F, D),
                         lambda s, tl, sl, a, r1, r2, wn: (sl[s], 0, 0)),
            pl.BlockSpec((1, F, D),
                         lambda s, tl, sl, a, r1, r2, wn: (sl[s], 0, 0)),
        ],
        out_specs=pl.BlockSpec((TM, D),
                               lambda s, tl, sl, a, r1, r2, wn: (tl[s], 0)),
    )
    return pl.pallas_call(
        _experts_body,
        grid_spec=grid_spec,
        out_shape=jax.ShapeDtypeStruct((M, D), jnp.float32),
    )(tl, sel, act_k, rs_k, re_k, wn_k, xs, w1, w3, w2)


def _combine(ys, p1, p2, wa, wb, i1, i2, offsets):
    mesh = plsc.VectorSubcoreMesh(core_axis_name="c", subcore_axis_name="s")

    @functools.partial(
        pl.kernel,
        mesh=mesh,
        out_type=jax.ShapeDtypeStruct((T, D), jnp.float32),
        scratch_types=[
            pltpu.VMEM((CT2, D), jnp.float32),
            pltpu.VMEM((CT2, D), jnp.float32),
            pltpu.VMEM((CT2, D), jnp.float32),
            pltpu.VMEM((CT2, D), jnp.float32),
            pltpu.VMEM((CT2, D), jnp.float32),
            pltpu.VMEM((CT2,), jnp.int32),
            pltpu.VMEM((CT2,), jnp.int32),
            pltpu.VMEM((CT2,), jnp.int32),
            pltpu.VMEM((CT2,), jnp.int32),
            pltpu.VMEM((CT2,), jnp.int32),
            pltpu.VMEM((CT2,), jnp.float32),
            pltpu.VMEM((CT2,), jnp.float32),
            pltpu.VMEM((CT2,), jnp.float32),
            pltpu.VMEM((CT2,), jnp.float32),
            pltpu.VMEM((16,), jnp.int32),
            pltpu.SemaphoreType.DMA,
            pltpu.SemaphoreType.DMA,
        ],
        compiler_params=pltpu.CompilerParams(needs_layout_passes=False),
    )
    def k(ys_hbm, p1_hbm, p2_hbm, wa_hbm, wb_hbm, i1_hbm, i2_hbm, off_hbm,
          out_hbm, aba, abb, bba, bbb, ob, q1a, q1b, q2a, q2b, e1,
          vaa, vab, vba, vbb, offs, sem1, sem2):
        wid = lax.axis_index("s") * NC + lax.axis_index("c")
        offs[...] = jnp.zeros((16,), jnp.int32)
        pltpu.sync_copy(off_hbm, offs.at[pl.ds(0, E)])

        ab = (aba, abb)
        bb = (bba, bbb)
        q1 = (q1a, q1b)
        q2 = (q2a, q2b)
        va = (vaa, vab)
        vb = (vba, vbb)
        sems = (sem1, sem2)
        nchunk = TPW // CT2

        def load_idx(c, b):
            tok0 = wid * TPW + c * CT2
            pltpu.sync_copy(p1_hbm.at[pl.ds(tok0, CT2)], q1[b])
            pltpu.sync_copy(i1_hbm.at[pl.ds(tok0, CT2)], e1)
            for v in range(CT2 // 16):
                sl = pl.ds(v * 16, 16)
                q1[b][sl] = q1[b][sl] + plsc.load_gather(offs, [e1[sl]])
            pltpu.sync_copy(p2_hbm.at[pl.ds(tok0, CT2)], q2[b])
            pltpu.sync_copy(i2_hbm.at[pl.ds(tok0, CT2)], e1)
            for v in range(CT2 // 16):
                sl = pl.ds(v * 16, 16)
                q2[b][sl] = q2[b][sl] + plsc.load_gather(offs, [e1[sl]])
            pltpu.sync_copy(wa_hbm.at[pl.ds(tok0, CT2)], va[b])
            pltpu.sync_copy(wb_hbm.at[pl.ds(tok0, CT2)], vb[b])

        def start_gathers(b):
            ca = pltpu.async_copy(ys_hbm.at[q1[b]], ab[b], sems[b])
            cb = pltpu.async_copy(ys_hbm.at[q2[b]], bb[b], sems[b])
            return ca, cb

        load_idx(0, 0)
        pend = start_gathers(0)
        for c in range(nchunk):
            b = c % 2
            nb = 1 - b
            if c + 1 < nchunk:
                load_idx(c + 1, nb)
            ca, cb = pend
            ca.wait()
            cb.wait()
            if c + 1 < nchunk:
                pend = start_gathers(nb)

            def tok(j, _, _b=b):
                j16 = jnp.full((16,), j, jnp.int32)
                wa16 = plsc.load_gather(va[_b], [j16])
                wb16 = plsc.load_gather(vb[_b], [j16])
                for cc in range(D // 16):
                    sl = pl.ds(cc * 16, 16)
                    ob[j, sl] = wa16 * ab[_b][j, sl] + wb16 * bb[_b][j, sl]
                return 0

            lax.fori_loop(0, CT2, tok, 0)
            tok0 = wid * TPW + c * CT2
            pltpu.sync_copy(ob, out_hbm.at[pl.ds(tok0, CT2)])

    return k(ys, p1, p2, wa, wb, i1, i2, offsets)


@jax.jit
def kernel(x, gate_w, w1, w2, w3):
    gl = x @ gate_w.T
    i1, i2, wa, wb, p1, p2, counts, offs = _router(gl)
    i1f = i1.reshape(T)
    i2f = i2.reshape(T)
    p1f = p1.reshape(T)
    p2f = p2.reshape(T)
    offf = offs.reshape(E)
    xs = _dispatch(x, i1f, i2f, p1f, p2f, offf)
    ys = _experts(xs, w1, w2, w3, counts)
    out = _combine(ys, p1f, p2f, wa.reshape(T), wb.reshape(T), i1f, i2f, offf)
    return out


# speedup vs baseline: 1.1111x; 1.0039x over previous
import functools

import jax
import jax.numpy as jnp
from jax import lax
from jax.experimental import pallas as pl
from jax.experimental.pallas import tpu as pltpu
from jax.experimental.pallas import tpu_sc as plsc

T = 8192
D = 768
F = 1024
E = 8
K = 2

TB = 1024
TM = 512
M = T * K
NT = M // TM

NC = 2
NS = 16
NW = NC * NS
TPW = T // NW
CT = 64
CT2 = 32


def _router_body(gl_ref, i1_ref, i2_ref, wa_ref, wb_ref,
                 p1_ref, p2_ref, cnt_ref, off_ref, carry_ref):
    b = pl.program_id(0)

    @pl.when(b == 0)
    def _():
        carry_ref[...] = jnp.zeros_like(carry_ref)

    logits = gl_ref[...]
    eids = lax.broadcasted_iota(jnp.int32, (TB, E), 1)
    m1 = jnp.max(logits, axis=1, keepdims=True)
    i1 = jnp.min(jnp.where(logits == m1, eids, E), axis=1)[:, None]
    oh1 = (eids == i1).astype(jnp.float32)
    masked = jnp.where(eids == i1, -jnp.inf, logits)
    m2 = jnp.max(masked, axis=1, keepdims=True)
    i2 = jnp.min(jnp.where(masked == m2, eids, E), axis=1)[:, None]
    oh2 = (eids == i2).astype(jnp.float32)

    t = jnp.exp(m2 - m1)
    wa = 1.0 / (1.0 + t)
    wb = 1.0 - wa

    ohs = oh1 + oh2
    r_i = lax.broadcasted_iota(jnp.int32, (TB, TB), 0)
    c_i = lax.broadcasted_iota(jnp.int32, (TB, TB), 1)
    ltri = (c_i < r_i).astype(jnp.float32)
    s_excl = lax.dot_general(ltri, ohs, (((1,), (0,)), ((), ())),
                             preferred_element_type=jnp.float32)
    base = carry_ref[...] + s_excl
    p1 = jnp.sum(oh1 * base, axis=1)[:, None]
    p2 = jnp.sum(oh2 * (base + oh1), axis=1)[:, None]
    new_carry = carry_ref[...][-1:] + jnp.sum(ohs, axis=0, keepdims=True)
    carry_ref[...] = jnp.broadcast_to(new_carry, (TB, E))

    i1_ref[...] = i1
    i2_ref[...] = i2
    wa_ref[...] = wa
    wb_ref[...] = wb
    p1_ref[...] = p1.astype(jnp.int32)
    p2_ref[...] = p2.astype(jnp.int32)
    cnt_ref[...] = new_carry.astype(jnp.int32)
    s = new_carry
    s = s + jnp.concatenate([jnp.zeros((1, 1), s.dtype), s[:, :-1]], axis=1)
    s = s + jnp.concatenate([jnp.zeros((1, 2), s.dtype), s[:, :-2]], axis=1)
    s = s + jnp.concatenate([jnp.zeros((1, 4), s.dtype), s[:, :-4]], axis=1)
    off_ref[...] = (s - new_carry).astype(jnp.int32)


def _router(gl):
    nb = T // TB
    out_shapes = (
        jax.ShapeDtypeStruct((T, 1), jnp.int32),
        jax.ShapeDtypeStruct((T, 1), jnp.int32),
        jax.ShapeDtypeStruct((T, 1), jnp.float32),
        jax.ShapeDtypeStruct((T, 1), jnp.float32),
        jax.ShapeDtypeStruct((T, 1), jnp.int32),
        jax.ShapeDtypeStruct((T, 1), jnp.int32),
        jax.ShapeDtypeStruct((1, E), jnp.int32),
        jax.ShapeDtypeStruct((1, E), jnp.int32),
    )
    tok_spec = pl.BlockSpec((TB, 1), lambda b: (b, 0))
    return pl.pallas_call(
        _router_body,
        grid=(nb,),
        in_specs=[
            pl.BlockSpec((TB, E), lambda b: (b, 0)),
        ],
        out_specs=(tok_spec, tok_spec, tok_spec, tok_spec, tok_spec, tok_spec,
                   pl.BlockSpec((1, E), lambda b: (0, 0)),
                   pl.BlockSpec((1, E), lambda b: (0, 0))),
        out_shape=out_shapes,
        scratch_shapes=[pltpu.VMEM((TB, E), jnp.float32)],
    )(gl)


def _dispatch(x, i1, i2, p1, p2, offsets):
    mesh = plsc.VectorSubcoreMesh(core_axis_name="c", subcore_axis_name="s")

    @functools.partial(
        pl.kernel,
        mesh=mesh,
        out_type=jax.ShapeDtypeStruct((M, D), jnp.float32),
        scratch_types=[
            pltpu.VMEM((CT, D), jnp.float32),
            pltpu.VMEM((CT, D), jnp.float32),
            pltpu.VMEM((CT,), jnp.int32),
            pltpu.VMEM((CT,), jnp.int32),
            pltpu.VMEM((CT,), jnp.int32),
            pltpu.VMEM((CT,), jnp.int32),
            pltpu.VMEM((CT,), jnp.int32),
            pltpu.VMEM((16,), jnp.int32),
            pltpu.SemaphoreType.DMA,
            pltpu.SemaphoreType.DMA,
            pltpu.SemaphoreType.DMA,
        ],
        compiler_params=pltpu.CompilerParams(needs_layout_passes=False),
    )
    def k(x_hbm, i1_hbm, i2_hbm, p1_hbm, p2_hbm, off_hbm, xs_hbm,
          xba, xbb, q1a, q1b, q2a, q2b, eb, offs, semx, sema, semb):
        wid = lax.axis_index("s") * NC + lax.axis_index("c")
        offs[...] = jnp.zeros((16,), jnp.int32)
        pltpu.sync_copy(off_hbm, offs.at[pl.ds(0, E)])

        xb = (xba, xbb)
        q1 = (q1a, q1b)
        q2 = (q2a, q2b)
        sems = (sema, semb)
        nchunk = TPW // CT

        def load_idx(c, b):
            tok0 = wid * TPW + c * CT
            pltpu.sync_copy(i1_hbm.at[pl.ds(tok0, CT)], eb)
            pltpu.sync_copy(p1_hbm.at[pl.ds(tok0, CT)], q1[b])
            for v in range(CT // 16):
                sl = pl.ds(v * 16, 16)
                q1[b][sl] = q1[b][sl] + plsc.load_gather(offs, [eb[sl]])
            pltpu.sync_copy(i2_hbm.at[pl.ds(tok0, CT)], eb)
            pltpu.sync_copy(p2_hbm.at[pl.ds(tok0, CT)], q2[b])
            for v in range(CT // 16):
                sl = pl.ds(v * 16, 16)
                q2[b][sl] = q2[b][sl] + plsc.load_gather(offs, [eb[sl]])

        cpx = pltpu.async_copy(x_hbm.at[pl.ds(wid * TPW, CT)], xba, semx)
        load_idx(0, 0)
        cpx.wait()
        pend = None
        for c in range(nchunk):
            b = c % 2
            nb = 1 - b
            if c + 1 < nchunk:
                tok0n = wid * TPW + (c + 1) * CT
                cpx = pltpu.async_copy(x_hbm.at[pl.ds(tok0n, CT)], xb[nb],
                                       semx)
            if pend is not None:
                pend[0].wait()
                pend[1].wait()
            s1 = pltpu.async_copy(xb[b], xs_hbm.at[q1[b]], sems[b])
            s2 = pltpu.async_copy(xb[b], xs_hbm.at[q2[b]], sems[b])
            pend = (s1, s2)
            if c + 1 < nchunk:
                load_idx(c + 1, nb)
                cpx.wait()
        pend[0].wait()
        pend[1].wait()

    return k(x, i1, i2, p1, p2, offsets)


NS_ITEMS = NT + E - 1


def _experts_body(tl_ref, sel_ref, act_ref, rs_ref, re_ref, wn_ref,
                  xs_ref, w1_ref, w3_ref, w2_ref, out_ref):
    s = pl.program_id(0)

    @pl.when(act_ref[s] == 1)
    def _():
        xb = xs_ref[...].astype(jnp.bfloat16)
        a = lax.dot_general(xb, w1_ref[0].astype(jnp.bfloat16),
                            (((1,), (1,)), ((), ())),
                            preferred_element_type=jnp.float32)
        g = lax.dot_general(xb, w3_ref[0].astype(jnp.bfloat16),
                            (((1,), (1,)), ((), ())),
                            preferred_element_type=jnp.float32)
        h = ((a / (1.0 + jnp.exp(-a))) * g).astype(jnp.bfloat16)
        y = lax.dot_general(h, w2_ref[0].astype(jnp.bfloat16),
                            (((1,), (0,)), ((), ())),
                            preferred_element_type=jnp.float32)
        rows = lax.broadcasted_iota(jnp.int32, (TM, 1), 0)
        mask = (rows >= rs_ref[s]) & (rows < re_ref[s])
        out_ref[...] = jnp.where(mask, y, out_ref[...])


def _experts(xs, w1, w2, w3, counts):
    cnts = counts.reshape(E)
    off = jnp.concatenate([jnp.zeros((1,), jnp.int32), jnp.cumsum(cnts)])
    t0 = jnp.arange(NT, dtype=jnp.int32) * TM
    sg = jnp.maximum(off[:E][None, :], t0[:, None])
    eg = jnp.minimum(off[1:][None, :], t0[:, None] + TM)
    act = (eg > sg).astype(jnp.int32)
    rsf = (sg - t0[:, None]).reshape(-1)
    ref_ = (eg - t0[:, None]).reshape(-1)
    afl = act.reshape(-1)
    csum = jnp.cumsum(afl)
    n_items = csum[-1]
    k = jnp.arange(NS_ITEMS, dtype=jnp.int32)
    target = jnp.minimum(k + 1, n_items)
    s_k = jnp.searchsorted(csum, target, side="left").astype(jnp.int32)
    tl = s_k // E
    sel = s_k % E
    act_k = (k < n_items).astype(jnp.int32)
    rs_k = jnp.where(act_k == 1, rsf[s_k], 0).astype(jnp.int32)
    re_k = jnp.where(act_k == 1, ref_[s_k], 0).astype(jnp.int32)
    wn_k = jnp.where(
        k == 0, 1,
        (sel != jnp.roll(sel, 1)).astype(jnp.int32)).astype(jnp.int32)

    grid_spec = pltpu.PrefetchScalarGridSpec(
        num_scalar_prefetch=6,
        grid=(NS_ITEMS,),
        in_specs=[
            pl.BlockSpec((TM, D),
                         lambda s, tl, sl, a, r1, r2, wn: (tl[s], 0)),
            pl.BlockSpec((1, F, D),
                         lambda s, tl, sl, a, r1, r2, wn: (sl[s], 0, 0)),
            pl.BlockSpec((1, F, D),
                         lambda s, tl, sl, a, r1, r2, wn: (sl[s], 0, 0)),
            pl.BlockSpec((1, F, D),
                         lambda s, tl, sl, a, r1, r2, wn: (sl[s], 0, 0)),
        ],
        out_specs=pl.BlockSpec((TM, D),
                               lambda s, tl, sl, a, r1, r2, wn: (tl[s], 0)),
    )
    return pl.pallas_call(
        _experts_body,
        grid_spec=grid_spec,
        out_shape=jax.ShapeDtypeStruct((M, D), jnp.float32),
    )(tl, sel, act_k, rs_k, re_k, wn_k, xs, w1, w3, w2)


def _combine(ys, p1, p2, wa, wb, i1, i2, offsets):
    mesh = plsc.VectorSubcoreMesh(core_axis_name="c", subcore_axis_name="s")

    @functools.partial(
        pl.kernel,
        mesh=mesh,
        out_type=jax.ShapeDtypeStruct((T, D), jnp.float32),
        scratch_types=[
            pltpu.VMEM((CT2, D), jnp.float32),
            pltpu.VMEM((CT2, D), jnp.float32),
            pltpu.VMEM((CT2, D), jnp.float32),
            pltpu.VMEM((CT2, D), jnp.float32),
            pltpu.VMEM((CT2, D), jnp.float32),
            pltpu.VMEM((CT2,), jnp.int32),
            pltpu.VMEM((CT2,), jnp.int32),
            pltpu.VMEM((CT2,), jnp.int32),
            pltpu.VMEM((CT2,), jnp.int32),
            pltpu.VMEM((CT2,), jnp.int32),
            pltpu.VMEM((CT2,), jnp.float32),
            pltpu.VMEM((CT2,), jnp.float32),
            pltpu.VMEM((CT2,), jnp.float32),
            pltpu.VMEM((CT2,), jnp.float32),
            pltpu.VMEM((16,), jnp.int32),
            pltpu.SemaphoreType.DMA,
            pltpu.SemaphoreType.DMA,
        ],
        compiler_params=pltpu.CompilerParams(needs_layout_passes=False),
    )
    def k(ys_hbm, p1_hbm, p2_hbm, wa_hbm, wb_hbm, i1_hbm, i2_hbm, off_hbm,
          out_hbm, aba, abb, bba, bbb, ob, q1a, q1b, q2a, q2b, e1,
          vaa, vab, vba, vbb, offs, sem1, sem2):
        wid = lax.axis_index("s") * NC + lax.axis_index("c")
        offs[...] = jnp.zeros((16,), jnp.int32)
        pltpu.sync_copy(off_hbm, offs.at[pl.ds(0, E)])

        ab = (aba, abb)
        bb = (bba, bbb)
        q1 = (q1a, q1b)
        q2 = (q2a, q2b)
        va = (vaa, vab)
        vb = (vba, vbb)
        sems = (sem1, sem2)
        nchunk = TPW // CT2

        def load_idx(c, b):
            tok0 = wid * TPW + c * CT2
            pltpu.sync_copy(p1_hbm.at[pl.ds(tok0, CT2)], q1[b])
            pltpu.sync_copy(i1_hbm.at[pl.ds(tok0, CT2)], e1)
            for v in range(CT2 // 16):
                sl = pl.ds(v * 16, 16)
                q1[b][sl] = q1[b][sl] + plsc.load_gather(offs, [e1[sl]])
            pltpu.sync_copy(p2_hbm.at[pl.ds(tok0, CT2)], q2[b])
            pltpu.sync_copy(i2_hbm.at[pl.ds(tok0, CT2)], e1)
            for v in range(CT2 // 16):
                sl = pl.ds(v * 16, 16)
                q2[b][sl] = q2[b][sl] + plsc.load_gather(offs, [e1[sl]])
            pltpu.sync_copy(wa_hbm.at[pl.ds(tok0, CT2)], va[b])
            pltpu.sync_copy(wb_hbm.at[pl.ds(tok0, CT2)], vb[b])

        def start_gathers(b):
            ca = pltpu.async_copy(ys_hbm.at[q1[b]], ab[b], sems[b])
            cb = pltpu.async_copy(ys_hbm.at[q2[b]], bb[b], sems[b])
            return ca, cb

        load_idx(0, 0)
        pend = start_gathers(0)
        for c in range(nchunk):
            b = c % 2
            nb = 1 - b
            if c + 1 < nchunk:
                load_idx(c + 1, nb)
            ca, cb = pend
            ca.wait()
            cb.wait()
            if c + 1 < nchunk:
                pend = start_gathers(nb)

            def tok(j, _, _b=b):
                j16 = jnp.full((16,), j, jnp.int32)
                wa16 = plsc.load_gather(va[_b], [j16])
                wb16 = plsc.load_gather(vb[_b], [j16])
                for cc in range(D // 16):
                    sl = pl.ds(cc * 16, 16)
                    ob[j, sl] = wa16 * ab[_b][j, sl] + wb16 * bb[_b][j, sl]
                return 0

            lax.fori_loop(0, CT2, tok, 0)
            tok0 = wid * TPW + c * CT2
            pltpu.sync_copy(ob, out_hbm.at[pl.ds(tok0, CT2)])

    return k(ys, p1, p2, wa, wb, i1, i2, offsets)


@jax.jit
def kernel(x, gate_w, w1, w2, w3):
    gl = x @ gate_w.T
    i1, i2, wa, wb, p1, p2, counts, offs = _router(gl)
    i1f = i1.reshape(T)
    i2f = i2.reshape(T)
    p1f = p1.reshape(T)
    p2f = p2.reshape(T)
    offf = offs.reshape(E)
    xs = _dispatch(x, i1f, i2f, p1f, p2f, offf)
    ys = _experts(xs, w1, w2, w3, counts)
    out = _combine(ys, p1f, p2f, wa.reshape(T), wb.reshape(T), i1f, i2f, offf)
    return out
